# Initial kernel scaffold; baseline (speedup 1.0000x reference)
import jax
import jax.numpy as jnp
from jax.experimental import pallas as pl


def kernel(x, edge_index, W0, b0, W1, b1):
    raise NotImplementedError("write your pallas kernel here")


# speedup vs baseline: 10.1764x; 10.1764x over previous
import functools

import jax
import jax.numpy as jnp
from jax import lax
from jax.experimental import pallas as pl
from jax.experimental.pallas import tpu as pltpu
from jax.experimental.pallas import tpu_sc as plsc

NC = 2
NS = 16
LANES = 16
K = 128
TCB = 400


def _vector_mesh():
    return plsc.VectorSubcoreMesh(core_axis_name="c", subcore_axis_name="s")


def _fill(ref, rows, width, value):
    @pl.loop(0, rows)
    def _(i):
        @pl.loop(0, width, step=LANES)
        def _(j):
            ref.at[i].at[pl.ds(j, LANES)][...] = jnp.full((LANES,), value,
                                                          jnp.float32)


def _deg_call(dst_pad, npad, width):
    epad = dst_pad.shape[0]
    per_core = epad // NC
    per_sub = per_core // NS
    n_chunks = per_sub // K
    stripe = npad // NS

    @functools.partial(
        pl.kernel,
        out_type=jax.ShapeDtypeStruct((NC * npad, width), jnp.float32),
        mesh=_vector_mesh(),
        scratch_types=[
            pltpu.VMEM((K,), jnp.int32),
            pltpu.VMEM((K, width), jnp.float32),
            pltpu.VMEM_SHARED((npad, width), jnp.float32),
        ],
    )
    def k(dst_hbm, out_hbm, idx_v, ones_v, cnt_sh):
        cid = lax.axis_index("c")
        sid = lax.axis_index("s")
        _fill(ones_v, K, width, 0.0)
        n_full = stripe // K
        tail = stripe - n_full * K

        @pl.loop(0, n_full)
        def _(t):
            pltpu.sync_copy(ones_v, cnt_sh.at[pl.ds(sid * stripe + t * K, K)])
        if tail:
            pltpu.sync_copy(ones_v.at[pl.ds(0, tail)],
                            cnt_sh.at[pl.ds(sid * stripe + n_full * K, tail)])

        _fill(ones_v, K, width, 1.0)

        plsc.subcore_barrier()

        @pl.loop(0, n_chunks)
        def _(ci):
            base = cid * per_core + sid * per_sub + ci * K
            pltpu.sync_copy(dst_hbm.at[pl.ds(base, K)], idx_v)
            pltpu.sync_copy(ones_v, cnt_sh.at[idx_v], add=True)

        plsc.subcore_barrier()

        @pl.loop(0, n_full)
        def _(t):
            r = sid * stripe + t * K
            pltpu.sync_copy(cnt_sh.at[pl.ds(r, K)],
                            out_hbm.at[pl.ds(cid * npad + r, K)])
        if tail:
            r = sid * stripe + n_full * K
            pltpu.sync_copy(cnt_sh.at[pl.ds(r, tail)],
                            out_hbm.at[pl.ds(cid * npad + r, tail)])

    return k(dst_pad)


def _agg_call(p, src_pad, dst_pad, npad, width):
    epad = src_pad.shape[0]
    per_core = epad // NC
    per_sub = per_core // NS
    n_chunks = per_sub // K
    stripe = npad // NS

    @functools.partial(
        pl.kernel,
        out_type=jax.ShapeDtypeStruct((NC * npad, width), jnp.float32),
        mesh=_vector_mesh(),
        scratch_types=[
            pltpu.VMEM((K,), jnp.int32),
            pltpu.VMEM((K,), jnp.int32),
            pltpu.VMEM((K, width), jnp.float32),
            pltpu.VMEM_SHARED((npad, width), jnp.float32),
            pltpu.SemaphoreType.DMA,
        ],
    )
    def k(p_hbm, src_hbm, dst_hbm, out_hbm, src_v, dst_v, rows_v, s_sh, sem):
        cid = lax.axis_index("c")
        sid = lax.axis_index("s")
        _fill(rows_v, K, width, 0.0)
        n_full = stripe // K
        tail = stripe - n_full * K

        @pl.loop(0, n_full)
        def _(t):
            pltpu.sync_copy(rows_v, s_sh.at[pl.ds(sid * stripe + t * K, K)])
        if tail:
            pltpu.sync_copy(rows_v.at[pl.ds(0, tail)],
                            s_sh.at[pl.ds(sid * stripe + n_full * K, tail)])

        plsc.subcore_barrier()

        @pl.loop(0, n_chunks)
        def _(ci):
            base = cid * per_core + sid * per_sub + ci * K
            pltpu.sync_copy(src_hbm.at[pl.ds(base, K)], src_v)
            pltpu.sync_copy(dst_hbm.at[pl.ds(base, K)], dst_v)
            pltpu.async_copy(p_hbm.at[src_v], rows_v, sem).wait()
            pltpu.sync_copy(rows_v, s_sh.at[dst_v], add=True)

        plsc.subcore_barrier()

        @pl.loop(0, n_full)
        def _(t):
            r = sid * stripe + t * K
            pltpu.sync_copy(s_sh.at[pl.ds(r, K)],
                            out_hbm.at[pl.ds(cid * npad + r, K)])
        if tail:
            r = sid * stripe + n_full * K
            pltpu.sync_copy(s_sh.at[pl.ds(r, tail)],
                            out_hbm.at[pl.ds(cid * npad + r, tail)])

    return k(p, src_pad, dst_pad)


def _dinv_block(c0, c1):
    deg = c0[:, 0] + c1[:, 0] + 1.0
    return lax.rsqrt(deg)


def _p0_call(x, w0, cnt, npad):
    n, d = x.shape
    h = w0.shape[1]

    def body(x_ref, w_ref, c0_ref, c1_ref, p_ref):
        dinv = _dinv_block(c0_ref, c1_ref)
        hw = jnp.dot(x_ref[...], w_ref[...], preferred_element_type=jnp.float32)
        p_ref[...] = hw * dinv[:, None]

    return pl.pallas_call(
        body,
        grid=(n // TCB,),
        in_specs=[
            pl.BlockSpec((TCB, d), lambda i: (i, 0)),
            pl.BlockSpec((d, h), lambda i: (0, 0)),
            pl.BlockSpec((TCB, 128), lambda i: (i, 0)),
            pl.BlockSpec((TCB, 128), lambda i: (i + npad // TCB, 0)),
        ],
        out_specs=pl.BlockSpec((TCB, h), lambda i: (i, 0)),
        out_shape=jax.ShapeDtypeStruct((n, h), jnp.float32),
    )(x, w0, cnt, cnt)


def _p1_call(s0, p0, cnt, b0, npad):
    n, h = p0.shape

    def body(s0a, s0b, p0_ref, c0_ref, c1_ref, b_ref, p1_ref):
        dinv = _dinv_block(c0_ref, c1_ref)
        hmat = (s0a[...] + s0b[...] + p0_ref[...]) * dinv[:, None] + b_ref[...]
        hmat = jnp.maximum(hmat, 0.0)
        p1_ref[...] = hmat * dinv[:, None]

    return pl.pallas_call(
        body,
        grid=(n // TCB,),
        in_specs=[
            pl.BlockSpec((TCB, h), lambda i: (i, 0)),
            pl.BlockSpec((TCB, h), lambda i: (i + npad // TCB, 0)),
            pl.BlockSpec((TCB, h), lambda i: (i, 0)),
            pl.BlockSpec((TCB, 128), lambda i: (i, 0)),
            pl.BlockSpec((TCB, 128), lambda i: (i + npad // TCB, 0)),
            pl.BlockSpec((1, h), lambda i: (0, 0)),
        ],
        out_specs=pl.BlockSpec((TCB, h), lambda i: (i, 0)),
        out_shape=jax.ShapeDtypeStruct((n, h), jnp.float32),
    )(s0, s0, p0, cnt, cnt, b0)


def _out_call(s1, p1, cnt, w1, b1, npad):
    n, h = p1.shape
    c = w1.shape[1]

    def body(s1a, s1b, p1_ref, c0_ref, c1_ref, w_ref, b_ref, o_ref):
        dinv = _dinv_block(c0_ref, c1_ref)
        agg = (s1a[...] + s1b[...] + p1_ref[...]) * dinv[:, None]
        o_ref[...] = jnp.dot(agg, w_ref[...],
                             preferred_element_type=jnp.float32) + b_ref[...]

    return pl.pallas_call(
        body,
        grid=(n // TCB,),
        in_specs=[
            pl.BlockSpec((TCB, h), lambda i: (i, 0)),
            pl.BlockSpec((TCB, h), lambda i: (i + npad // TCB, 0)),
            pl.BlockSpec((TCB, h), lambda i: (i, 0)),
            pl.BlockSpec((TCB, 128), lambda i: (i, 0)),
            pl.BlockSpec((TCB, 128), lambda i: (i + npad // TCB, 0)),
            pl.BlockSpec((h, c), lambda i: (0, 0)),
            pl.BlockSpec((1, c), lambda i: (0, 0)),
        ],
        out_specs=pl.BlockSpec((TCB, c), lambda i: (i, 0)),
        out_shape=jax.ShapeDtypeStruct((n, c), jnp.float32),
    )(s1, s1, p1, cnt, cnt, w1, b1)


def kernel(x, edge_index, W0, b0, W1, b1):
    n, d = x.shape
    h = W0.shape[1]

    src, dst = edge_index[0], edge_index[1]
    e = src.shape[0]
    chunk_total = NC * NS * K
    epad = ((e + chunk_total - 1) // chunk_total) * chunk_total
    npad = ((n + 3199) // 3200) * 3200

    pad = epad - e
    src_p = jnp.concatenate([src, jnp.zeros((pad,), src.dtype)])
    dst_p = jnp.concatenate([dst, jnp.full((pad,), n, dst.dtype)])
    b0r = b0.reshape(1, h)
    b1r = b1.reshape(1, b1.shape[0])

    cnt = _deg_call(dst_p, npad, h)
    p0 = _p0_call(x, W0, cnt, npad)
    s0 = _agg_call(p0, src_p, dst_p, npad, h)
    p1 = _p1_call(s0, p0, cnt, b0r, npad)
    s1 = _agg_call(p1, src_p, dst_p, npad, h)
    return _out_call(s1, p1, cnt, W1, b1r, npad)
